# baseline (speedup 1.0000x reference)
import functools

import jax
import jax.numpy as jnp
from jax import lax
from jax.experimental import pallas as pl
from jax.experimental.pallas import tpu as pltpu
from jax.experimental.pallas import tpu_sc as plsc

EMB = 128
OUT_D = 3 * EMB
NC = 2
NS = 16
NW = NC * NS
CHUNK = 128
NBUF = 2


def _tc_cost_body(cost_ref, wc_ref, bc_ref, out_ref):
    c = cost_ref[...]
    w = wc_ref[...][0]
    bb = bc_ref[...][0]
    out_ref[...] = c[..., None] * w[None, None, None, :] + bb[None, None, None, :]


def _sc_body(seq_h, pos_h, deg_h, wd_h, bd_h, table_h, out_h,
             idx_v, deg_v, rows_v, cd_v, pos0_v, pos1_v,
             wd_v, bd_v,
             sem_in0, sem_in1, sem_deg0, sem_deg1, sem_rows0, sem_rows1,
             sem_out0, sem_out1, sem_pos0, sem_pos1, *, per_w, nbatch, jk):
    wid = lax.axis_index("s") * NC + lax.axis_index("c")
    base = wid * per_w
    nchunk = per_w // CHUNK
    nhalf = nchunk // NBUF

    sem_in = [sem_in0, sem_in1]
    sem_deg = [sem_deg0, sem_deg1]
    sem_rows = [sem_rows0, sem_rows1]
    sem_out = [sem_out0, sem_out1]

    pltpu.sync_copy(wd_h, wd_v)
    pltpu.sync_copy(bd_h, bd_v)

    nj = EMB // 16
    wd_s = [wd_v[pl.ds(j * 16, 16)] for j in range(nj)]
    bd_s = [bd_v[pl.ds(j * 16, 16)] for j in range(nj)]

    pos_refs = [pos0_v, pos1_v]
    sem_pos = [sem_pos0, sem_pos1]

    def issue_pos(b, g):
        pltpu.async_copy(pos_h.at[pl.ds(base + g * CHUNK, CHUNK)], pos_refs[b], sem_pos[b])

    def wait_pos(b):
        pltpu.make_async_copy(pos_h.at[pl.ds(base, CHUNK)], pos_refs[b], sem_pos[b]).wait()

    def issue_in(b):
        pos = pos_refs[b]
        pltpu.async_copy(seq_h.at[pos], idx_v.at[b], sem_in[b])

    def wait_in(b):
        pos = pos_refs[b]
        pltpu.make_async_copy(seq_h.at[pos], idx_v.at[b], sem_in[b]).wait()

    def wait_out(b):
        pltpu.make_async_copy(cd_v.at[b], out_h.at[pl.ds(base, CHUNK), pl.ds(0, EMB)], sem_out[b]).wait()
        pltpu.make_async_copy(rows_v.at[b], out_h.at[pl.ds(base, CHUNK), pl.ds(2 * EMB, EMB)], sem_out[b]).wait()

    def compute(b):
        dv = deg_v.at[b]
        cd = cd_v.at[b]

        def grp_body(gi, c2):
            r0 = gi * 16
            deg16 = dv[pl.ds(r0, 16)].astype(jnp.float32)
            for k in range(16):
                d = deg16[k]
                row = r0 + k
                for j in range(nj):
                    cd[row, pl.ds(j * 16, 16)] = d * wd_s[j] + bd_s[j]
            return c2

        lax.fori_loop(0, CHUNK // 16, grp_body, 0)

    def half_step(gi, b):
        g = NBUF * gi + b
        off = base + g * CHUNK

        @pl.when(gi >= 1)
        def _():
            wait_out(b)

        wait_in(b)
        cp_deg = pltpu.async_copy(deg_h.at[idx_v.at[b]], deg_v.at[b], sem_deg[b])
        cp_rows = pltpu.async_copy(table_h.at[idx_v.at[b]], rows_v.at[b], sem_rows[b])
        @pl.when(gi < nhalf - 1)
        def _():
            issue_pos(b, g + 2)

        o = 1 - b
        if b == 0:
            wait_pos(o)
            issue_in(o)
        else:
            @pl.when(gi < nhalf - 1)
            def _():
                wait_pos(o)
                issue_in(o)

        cp_deg.wait()
        compute(b)
        pltpu.async_copy(cd_v.at[b], out_h.at[pl.ds(off, CHUNK), pl.ds(0, EMB)], sem_out[b])
        cp_rows.wait()
        pltpu.async_copy(rows_v.at[b], out_h.at[pl.ds(off, CHUNK), pl.ds(2 * EMB, EMB)], sem_out[b])

    issue_pos(0, 0)
    issue_pos(1, 1)
    wait_pos(0)
    issue_in(0)

    def loop_body(gi, carry):
        half_step(gi, 0)
        half_step(gi, 1)
        return carry

    lax.fori_loop(0, nhalf, loop_body, 0)
    wait_out(0)
    wait_out(1)


def kernel(sequence, cost, degrees, W_cost, b_cost, W_deg, b_deg, node_table):
    b, num_walks, len_walk = sequence.shape
    jk = num_walks * len_walk
    total = b * jk
    per_w = total // NW

    seq_t = jnp.transpose(sequence, (1, 2, 0)).reshape(-1).astype(jnp.int32)
    e = jnp.arange(total, dtype=jnp.int32)
    pos_all = (e % jk) * b + e // jk
    deg1 = degrees.astype(jnp.int32)
    wd = W_deg[:, 0]
    wc2 = W_cost[:, 0][None, :]
    bc2 = b_cost[None, :]

    grid_rows = 128
    tc = pl.pallas_call(
        _tc_cost_body,
        grid=(b // grid_rows,),
        in_specs=[
            pl.BlockSpec((grid_rows, num_walks, len_walk), lambda i: (i, 0, 0)),
            pl.BlockSpec((1, EMB), lambda i: (0, 0)),
            pl.BlockSpec((1, EMB), lambda i: (0, 0)),
        ],
        out_specs=pl.BlockSpec((grid_rows, num_walks, len_walk, EMB), lambda i: (i, 0, 0, 1)),
        out_shape=jax.ShapeDtypeStruct((b, num_walks, len_walk, OUT_D), jnp.float32),
    )
    out4 = tc(cost, wc2, bc2)

    out_ref = jax.new_ref(out4.reshape(total, OUT_D))
    mesh = plsc.VectorSubcoreMesh(core_axis_name="c", subcore_axis_name="s")
    f = pl.kernel(
        functools.partial(_sc_body, per_w=per_w, nbatch=b, jk=jk),
        mesh=mesh,
        out_type=(),
        scratch_types=[
            pltpu.VMEM((NBUF, CHUNK), jnp.int32),
            pltpu.VMEM((NBUF, CHUNK), jnp.int32),
            pltpu.VMEM((NBUF, CHUNK, EMB), jnp.float32),
            pltpu.VMEM((NBUF, CHUNK, EMB), jnp.float32),
            pltpu.VMEM((CHUNK,), jnp.int32),
            pltpu.VMEM((CHUNK,), jnp.int32),
            pltpu.VMEM((EMB,), jnp.float32),
            pltpu.VMEM((EMB,), jnp.float32),
            pltpu.SemaphoreType.DMA,
            pltpu.SemaphoreType.DMA,
            pltpu.SemaphoreType.DMA,
            pltpu.SemaphoreType.DMA,
            pltpu.SemaphoreType.DMA,
            pltpu.SemaphoreType.DMA,
            pltpu.SemaphoreType.DMA,
            pltpu.SemaphoreType.DMA,
            pltpu.SemaphoreType.DMA,
            pltpu.SemaphoreType.DMA,
        ],
    )
    f(seq_t, pos_all, deg1, wd, b_deg, node_table, out_ref)
    out = jax.ref.freeze(out_ref)
    return out.reshape(b, num_walks, len_walk, OUT_D)


# speedup vs baseline: 1.1099x; 1.1099x over previous
import functools

import jax
import jax.numpy as jnp
from jax import lax
from jax.experimental import pallas as pl
from jax.experimental.pallas import tpu as pltpu
from jax.experimental.pallas import tpu_sc as plsc

EMB = 128
OUT_D = 3 * EMB
NC = 2
NS = 16
NW = NC * NS
CHUNK = 128
NBUF = 2


def _sc_body(seq_h, cost_h, pos_h, deg_h, wd_h, bd_h, wc_h, bc_h, table_h, out_h,
             idx_v, deg_v, cost_v, rows_v, cd_v, pos0_v, pos1_v,
             wd_v, bd_v, wc_v, bc_v,
             sem_in0, sem_in1, sem_deg0, sem_deg1, sem_rows0, sem_rows1,
             sem_out0, sem_out1, sem_pos0, sem_pos1, *, per_w, nbatch, jk):
    wid = lax.axis_index("s") * NC + lax.axis_index("c")
    base = wid * per_w
    nchunk = per_w // CHUNK
    nhalf = nchunk // NBUF

    sem_in = [sem_in0, sem_in1]
    sem_deg = [sem_deg0, sem_deg1]
    sem_rows = [sem_rows0, sem_rows1]
    sem_out = [sem_out0, sem_out1]

    pltpu.sync_copy(wd_h, wd_v)
    pltpu.sync_copy(bd_h, bd_v)
    pltpu.sync_copy(wc_h, wc_v)
    pltpu.sync_copy(bc_h, bc_v)

    nj = EMB // 16
    wd_s = [wd_v[pl.ds(j * 16, 16)] for j in range(nj)]
    bd_s = [bd_v[pl.ds(j * 16, 16)] for j in range(nj)]
    wc_s = [wc_v[pl.ds(j * 16, 16)] for j in range(nj)]
    bc_s = [bc_v[pl.ds(j * 16, 16)] for j in range(nj)]

    pos_refs = [pos0_v, pos1_v]
    sem_pos = [sem_pos0, sem_pos1]

    def issue_pos(b, g):
        pltpu.async_copy(pos_h.at[pl.ds(base + g * CHUNK, CHUNK)], pos_refs[b], sem_pos[b])

    def wait_pos(b):
        pltpu.make_async_copy(pos_h.at[pl.ds(base, CHUNK)], pos_refs[b], sem_pos[b]).wait()

    def issue_in(b):
        pos = pos_refs[b]
        pltpu.async_copy(seq_h.at[pos], idx_v.at[b], sem_in[b])
        pltpu.async_copy(cost_h.at[pos], cost_v.at[b], sem_in[b])

    def wait_in(b):
        pos = pos_refs[b]
        pltpu.make_async_copy(seq_h.at[pos], idx_v.at[b], sem_in[b]).wait()
        pltpu.make_async_copy(cost_h.at[pos], cost_v.at[b], sem_in[b]).wait()

    def wait_out(b):
        pltpu.make_async_copy(cd_v.at[b], out_h.at[pl.ds(base, CHUNK), pl.ds(0, 2 * EMB)], sem_out[b]).wait()
        pltpu.make_async_copy(rows_v.at[b], out_h.at[pl.ds(base, CHUNK), pl.ds(2 * EMB, EMB)], sem_out[b]).wait()

    def compute(b):
        dv = deg_v.at[b]
        cv_ref = cost_v.at[b]
        cd = cd_v.at[b]

        def grp_body(gi, c2):
            r0 = gi * 16
            deg16 = dv[pl.ds(r0, 16)].astype(jnp.float32)
            cost16 = cv_ref[pl.ds(r0, 16)]
            for k in range(16):
                d = deg16[k]
                cv = cost16[k]
                row = r0 + k
                for j in range(nj):
                    cd[row, pl.ds(j * 16, 16)] = d * wd_s[j] + bd_s[j]
                    cd[row, pl.ds(EMB + j * 16, 16)] = cv * wc_s[j] + bc_s[j]
            return c2

        lax.fori_loop(0, CHUNK // 16, grp_body, 0)

    def half_step(gi, b):
        g = NBUF * gi + b
        off = base + g * CHUNK

        @pl.when(gi >= 1)
        def _():
            wait_out(b)

        wait_in(b)
        cp_deg = pltpu.async_copy(deg_h.at[idx_v.at[b]], deg_v.at[b], sem_deg[b])
        cp_rows = pltpu.async_copy(table_h.at[idx_v.at[b]], rows_v.at[b], sem_rows[b])
        @pl.when(gi < nhalf - 1)
        def _():
            issue_pos(b, g + 2)

        o = 1 - b
        if b == 0:
            wait_pos(o)
            issue_in(o)
        else:
            @pl.when(gi < nhalf - 1)
            def _():
                wait_pos(o)
                issue_in(o)

        cp_deg.wait()
        compute(b)
        pltpu.async_copy(cd_v.at[b], out_h.at[pl.ds(off, CHUNK), pl.ds(0, 2 * EMB)], sem_out[b])
        cp_rows.wait()
        pltpu.async_copy(rows_v.at[b], out_h.at[pl.ds(off, CHUNK), pl.ds(2 * EMB, EMB)], sem_out[b])

    issue_pos(0, 0)
    issue_pos(1, 1)
    wait_pos(0)
    issue_in(0)

    def loop_body(gi, carry):
        half_step(gi, 0)
        half_step(gi, 1)
        return carry

    lax.fori_loop(0, nhalf, loop_body, 0)
    wait_out(0)
    wait_out(1)


def kernel(sequence, cost, degrees, W_cost, b_cost, W_deg, b_deg, node_table):
    b, num_walks, len_walk = sequence.shape
    jk = num_walks * len_walk
    total = b * jk
    per_w = total // NW

    seq_t = jnp.transpose(sequence, (1, 2, 0)).reshape(-1).astype(jnp.int32)
    cost_t = jnp.transpose(cost, (1, 2, 0)).reshape(-1).astype(jnp.float32)
    e = jnp.arange(total, dtype=jnp.int32)
    pos_all = (e % jk) * b + e // jk
    deg1 = degrees.astype(jnp.int32)
    wd = W_deg[:, 0]
    wc = W_cost[:, 0]

    mesh = plsc.VectorSubcoreMesh(core_axis_name="c", subcore_axis_name="s")
    f = pl.kernel(
        functools.partial(_sc_body, per_w=per_w, nbatch=b, jk=jk),
        mesh=mesh,
        out_type=jax.ShapeDtypeStruct((total, OUT_D), jnp.float32),
        scratch_types=[
            pltpu.VMEM((NBUF, CHUNK), jnp.int32),
            pltpu.VMEM((NBUF, CHUNK), jnp.int32),
            pltpu.VMEM((NBUF, CHUNK), jnp.float32),
            pltpu.VMEM((NBUF, CHUNK, EMB), jnp.float32),
            pltpu.VMEM((NBUF, CHUNK, 2 * EMB), jnp.float32),
            pltpu.VMEM((CHUNK,), jnp.int32),
            pltpu.VMEM((CHUNK,), jnp.int32),
            pltpu.VMEM((EMB,), jnp.float32),
            pltpu.VMEM((EMB,), jnp.float32),
            pltpu.VMEM((EMB,), jnp.float32),
            pltpu.VMEM((EMB,), jnp.float32),
            pltpu.SemaphoreType.DMA,
            pltpu.SemaphoreType.DMA,
            pltpu.SemaphoreType.DMA,
            pltpu.SemaphoreType.DMA,
            pltpu.SemaphoreType.DMA,
            pltpu.SemaphoreType.DMA,
            pltpu.SemaphoreType.DMA,
            pltpu.SemaphoreType.DMA,
            pltpu.SemaphoreType.DMA,
            pltpu.SemaphoreType.DMA,
        ],
    )
    out = f(seq_t, cost_t, pos_all, deg1, wd, b_deg, wc, b_cost, node_table)
    return out.reshape(b, num_walks, len_walk, OUT_D)
